# baseline (speedup 1.0000x reference)
import functools

import jax
import jax.numpy as jnp
from jax import lax
from jax.experimental import pallas as pl
from jax.experimental.pallas import tpu as pltpu
from jax.experimental.pallas import tpu_sc as plsc

N = 50000
E = 800000
NC = 2
NS = 16
L = 16
CHUNK = 160
SUB = 80
NSUB = CHUNK // SUB
NBUF = 5
EP = 819200
ROWS_PER_TILE = N // NS
OUT_RPT = 3128
OUT_LAST = N - (NS - 1) * OUT_RPT


def _spmm_sc(colsplit: bool):
  edges_per_tile = EP // NS if colsplit else EP // (NS * NC)
  nchunk = edges_per_tile // CHUNK

  mesh = plsc.VectorSubcoreMesh(
      core_axis_name="c", subcore_axis_name="s", num_cores=NC, num_subcores=NS)

  @functools.partial(
      pl.kernel,
      mesh=mesh,
      compiler_params=pltpu.CompilerParams(use_tc_tiling_on_sc=False),
      out_type=jax.ShapeDtypeStruct((NC * N, 32), jnp.float32),
      scratch_types=(
          [pltpu.VMEM((CHUNK,), jnp.int32) for _ in range(NBUF)]
          + [pltpu.VMEM((NSUB, SUB), jnp.int32) for _ in range(NBUF)]
          + [pltpu.VMEM((CHUNK,), jnp.float32) for _ in range(NBUF)]
          + [pltpu.VMEM((CHUNK, 32), jnp.float32) for _ in range(NBUF)]
          + [pltpu.VMEM_SHARED((N, 32), jnp.float32)]
          + [pltpu.SemaphoreType.DMA for _ in range(3 * NBUF)]
      ),
  )
  def spmm(dst_hbm, src_hbm, eval_hbm, x_hbm, out_hbm, *scratch):
    src_v = scratch[0:NBUF]
    dst_v = scratch[NBUF:2 * NBUF]
    val_v = scratch[2 * NBUF:3 * NBUF]
    rows_v = scratch[3 * NBUF:4 * NBUF]
    acc_sh = scratch[4 * NBUF]
    semi = scratch[4 * NBUF + 1:4 * NBUF + 1 + NBUF]
    semg = scratch[4 * NBUF + 1 + NBUF:4 * NBUF + 1 + 2 * NBUF]
    sems = scratch[4 * NBUF + 1 + 2 * NBUF:4 * NBUF + 1 + 3 * NBUF]

    c = lax.axis_index("c")
    s = lax.axis_index("s")

    @plsc.parallel_loop(0, CHUNK)
    def _(i):
      rows_v[0][i, pl.ds(0, L)] = jnp.zeros((L,), jnp.float32)
      rows_v[0][i, pl.ds(L, L)] = jnp.zeros((L,), jnp.float32)

    for z in range(ROWS_PER_TILE // CHUNK):
      pltpu.sync_copy(rows_v[0],
                      acc_sh.at[pl.ds(s * ROWS_PER_TILE + z * CHUNK, CHUNK)])
    zrem = ROWS_PER_TILE % CHUNK
    if zrem:
      pltpu.sync_copy(
          rows_v[0].at[pl.ds(0, zrem)],
          acc_sh.at[pl.ds(s * ROWS_PER_TILE + (ROWS_PER_TILE // CHUNK) * CHUNK, zrem)])
    plsc.subcore_barrier()

    if colsplit:
      tile_base = s * edges_per_tile
    else:
      tile_base = (c * NS + s) * edges_per_tile

    def fire_idx(g, b):
      e0 = tile_base + g * CHUNK
      pltpu.async_copy(src_hbm.at[pl.ds(e0, CHUNK)], src_v[b], semi[b])
      pltpu.async_copy(eval_hbm.at[pl.ds(e0, CHUNK)], val_v[b], semi[b])
      for j in range(NSUB):
        pltpu.async_copy(dst_hbm.at[pl.ds(e0 + j * SUB, SUB)], dst_v[b].at[j],
                         semi[b])

    def wait_idx(b):
      pltpu.make_async_copy(src_hbm.at[pl.ds(0, CHUNK)], src_v[b], semi[b]).wait()
      pltpu.make_async_copy(eval_hbm.at[pl.ds(0, CHUNK)], val_v[b], semi[b]).wait()
      for j in range(NSUB):
        pltpu.make_async_copy(dst_hbm.at[pl.ds(0, SUB)], dst_v[b].at[j],
                              semi[b]).wait()

    def transform(b):
      if colsplit:
        @plsc.parallel_loop(0, CHUNK, step=L)
        def _(k):
          src_v[b][pl.ds(k, L)] = src_v[b][pl.ds(k, L)] * 2 + c

    def fire_gather(b):
      for j in range(NSUB):
        pltpu.async_copy(x_hbm.at[src_v[b].at[pl.ds(j * SUB, SUB)]],
                         rows_v[b].at[pl.ds(j * SUB, SUB)], semg[b])

    def wait_gather(b):
      for j in range(NSUB):
        pltpu.make_async_copy(x_hbm.at[src_v[b].at[pl.ds(j * SUB, SUB)]],
                              rows_v[b].at[pl.ds(j * SUB, SUB)], semg[b]).wait()

    def scale(b):
      @plsc.parallel_loop(0, CHUNK, step=L)
      def _(g):
        vv = val_v[b][pl.ds(g, L)]
        for e0 in range(L):
          sp = jnp.broadcast_to(vv[e0], (L,))
          e = g + e0
          rows_v[b][e, pl.ds(0, L)] = rows_v[b][e, pl.ds(0, L)] * sp
          rows_v[b][e, pl.ds(L, L)] = rows_v[b][e, pl.ds(L, L)] * sp

    def fire_scatter(b):
      for j in range(NSUB):
        pltpu.async_copy(rows_v[b].at[pl.ds(j * SUB, SUB)],
                         acc_sh.at[dst_v[b].at[j]], sems[b], add=True)

    def wait_scatter(b):
      for j in range(NSUB):
        pltpu.make_async_copy(rows_v[b].at[pl.ds(j * SUB, SUB)],
                              acc_sh.at[dst_v[b].at[j]], sems[b]).wait()

    for p in range(3):
      fire_idx(jnp.int32(p), p)
    for p in range(3):
      wait_idx(p)
      transform(p)
      fire_gather(p)
    fire_idx(jnp.int32(3), 3)

    def group_body(g0, _):
      for b in range(NBUF):
        g = g0 * NBUF + b
        b3 = (b + 3) % NBUF
        b4 = (b + 4) % NBUF
        wait_gather(b)
        scale(b)
        fire_scatter(b)

        @pl.when(g + 3 < nchunk)
        def _():
          wait_idx(b3)
          transform(b3)

        @pl.when(g >= 1)
        def _():
          wait_scatter(b4)

        @pl.when(g + 4 < nchunk)
        def _():
          fire_idx(g + 4, b4)

        @pl.when(g + 3 < nchunk)
        def _():
          fire_gather(b3)
      return 0

    lax.fori_loop(0, nchunk // NBUF, group_body, 0)
    wait_scatter((nchunk - 1) % NBUF)

    plsc.subcore_barrier()
    @pl.when(s < NS - 1)
    def _():
      pltpu.sync_copy(acc_sh.at[pl.ds(s * OUT_RPT, OUT_RPT)],
                      out_hbm.at[pl.ds(c * N + s * OUT_RPT, OUT_RPT)])

    @pl.when(s == NS - 1)
    def _():
      pltpu.sync_copy(acc_sh.at[pl.ds((NS - 1) * OUT_RPT, OUT_LAST)],
                      out_hbm.at[pl.ds(c * N + (NS - 1) * OUT_RPT, OUT_LAST)])

  return spmm


_spmm_col = _spmm_sc(True)
_spmm_edge = _spmm_sc(False)


def _leaky(x):
  return jnp.where(x >= 0, x, 0.01 * x)


def _bi_combine(ego, side, W1, b1, W2, b2):
  s1 = _leaky(jnp.dot(ego + side, W1, preferred_element_type=jnp.float32) + b1)
  s2 = _leaky(jnp.dot(ego * side, W2, preferred_element_type=jnp.float32) + b2)
  new = s1 + s2
  nrm = jnp.sqrt(jnp.sum(new * new, axis=-1, keepdims=True))
  return new, new / jnp.maximum(nrm, 1e-12)


BN = 2000


def _combine_a_body(ego_ref, side_ref, w1_ref, b1_ref, w2_ref, b2_ref,
                    nxt_ref, nrm_ref):
  ego = ego_ref[...].reshape(BN, 64)
  side = jnp.concatenate([side_ref[0], side_ref[1]], axis=-1)
  new, nn = _bi_combine(ego, side, w1_ref[...], b1_ref[...], w2_ref[...], b2_ref[...])
  nxt_ref[...] = new.reshape(BN, 2, 32)
  nrm_ref[...] = nn


def _combine_b_body(ego_ref, side_ref, w1_ref, b1_ref, w2_ref, b2_ref,
                    nxt_ref):
  ego = ego_ref[...].reshape(BN, 64)
  side = jnp.concatenate([side_ref[0], side_ref[1]], axis=-1)
  new, _ = _bi_combine(ego, side, w1_ref[...], b1_ref[...], w2_ref[...], b2_ref[...])
  nxt_ref[...] = new


def _combine_c_body(ego0_ref, nrm1_ref, ego2_ref, side_ref,
                    w1_ref, b1_ref, w2_ref, b2_ref, out_ref):
  ego2 = ego2_ref[...]
  n2 = jnp.sqrt(jnp.sum(ego2 * ego2, axis=-1, keepdims=True))
  nrm2 = ego2 / jnp.maximum(n2, 1e-12)
  side = side_ref[0] + side_ref[1]
  _, nrm3 = _bi_combine(ego2, side, w1_ref[...], b1_ref[...], w2_ref[...], b2_ref[...])
  out_ref[...] = jnp.concatenate([ego0_ref[...], nrm1_ref[...], nrm2, nrm3],
                                 axis=-1)


def _il_spec():
  return pl.BlockSpec((BN, 2, 32), lambda i: (i, 0, 0))


def _split_spec():
  return pl.BlockSpec((2, BN, 32), lambda i: (0, i, 0))


def _full_spec(d):
  return pl.BlockSpec((BN, d), lambda i: (i, 0))


def _w_spec(din, dout):
  return pl.BlockSpec((din, dout), lambda i: (0, 0))


def _b_spec(dout):
  return pl.BlockSpec((1, dout), lambda i: (0, 0))


def _combine_a(ego_il, side_sp, W1, b1, W2, b2):
  return pl.pallas_call(
      _combine_a_body,
      grid=(N // BN,),
      in_specs=[_il_spec(), _split_spec(), _w_spec(64, 64), _b_spec(64),
                _w_spec(64, 64), _b_spec(64)],
      out_specs=[_il_spec(), _full_spec(64)],
      out_shape=[jax.ShapeDtypeStruct((N, 2, 32), jnp.float32),
                 jax.ShapeDtypeStruct((N, 64), jnp.float32)],
  )(ego_il, side_sp, W1, b1.reshape(1, -1), W2, b2.reshape(1, -1))


def _combine_b(ego_il, side_sp, W1, b1, W2, b2):
  return pl.pallas_call(
      _combine_b_body,
      grid=(N // BN,),
      in_specs=[_il_spec(), _split_spec(), _w_spec(64, 32), _b_spec(32),
                _w_spec(64, 32), _b_spec(32)],
      out_specs=_full_spec(32),
      out_shape=jax.ShapeDtypeStruct((N, 32), jnp.float32),
  )(ego_il, side_sp, W1, b1.reshape(1, -1), W2, b2.reshape(1, -1))


def _combine_c(ego0, nrm1, ego2, side_sp, W1, b1, W2, b2):
  return pl.pallas_call(
      _combine_c_body,
      grid=(N // BN,),
      in_specs=[_full_spec(64), _full_spec(64), _full_spec(32), _split_spec(),
                _w_spec(32, 16), _b_spec(16), _w_spec(32, 16), _b_spec(16)],
      out_specs=_full_spec(176),
      out_shape=jax.ShapeDtypeStruct((N, 176), jnp.float32),
  )(ego0, nrm1, ego2, side_sp, W1, b1.reshape(1, -1), W2, b2.reshape(1, -1))


def kernel(ego_embeddings, edge_index, edge_values,
           W1_0, b1_0, W2_0, b2_0,
           W1_1, b1_1, W2_1, b2_1,
           W1_2, b1_2, W2_2, b2_2):
  dst = jnp.pad(edge_index[0], (0, EP - E))
  src = jnp.pad(edge_index[1], (0, EP - E))
  ev = jnp.pad(edge_values, (0, EP - E))

  x0 = ego_embeddings.reshape(2 * N, 32)
  side0 = _spmm_col(dst, src, ev, x0)
  ego1_il, norm1 = _combine_a(ego_embeddings.reshape(N, 2, 32),
                              side0.reshape(2, N, 32), W1_0, b1_0, W2_0, b2_0)

  side1 = _spmm_col(dst, src, ev, ego1_il.reshape(2 * N, 32))
  ego2 = _combine_b(ego1_il, side1.reshape(2, N, 32), W1_1, b1_1, W2_1, b2_1)

  side2 = _spmm_edge(dst, src, ev, ego2)
  return _combine_c(ego_embeddings, norm1, ego2, side2.reshape(2, N, 32),
                    W1_2, b1_2, W2_2, b2_2)


# speedup vs baseline: 1.8127x; 1.8127x over previous
import functools

import jax
import jax.numpy as jnp
from jax import lax
from jax.experimental import pallas as pl
from jax.experimental.pallas import tpu as pltpu
from jax.experimental.pallas import tpu_sc as plsc

N = 50000
E = 800000
NC = 2
NS = 16
L = 16
CHUNK = 216
SUB = 72
NSUB = CHUNK // SUB
NBUF = 4
EP = 801792
ROWS_PER_TILE = N // NS
OUT_RPT = 3128
OUT_LAST = N - (NS - 1) * OUT_RPT


def _spmm_sc(colsplit: bool):
  edges_per_tile = EP // NS if colsplit else EP // (NS * NC)
  nchunk = edges_per_tile // CHUNK

  mesh = plsc.VectorSubcoreMesh(
      core_axis_name="c", subcore_axis_name="s", num_cores=NC, num_subcores=NS)

  @functools.partial(
      pl.kernel,
      mesh=mesh,
      compiler_params=pltpu.CompilerParams(use_tc_tiling_on_sc=False),
      out_type=jax.ShapeDtypeStruct((NC * N, 32), jnp.float32),
      scratch_types=(
          [pltpu.VMEM((CHUNK,), jnp.int32) for _ in range(NBUF)]
          + [pltpu.VMEM((NSUB, SUB), jnp.int32) for _ in range(NBUF)]
          + [pltpu.VMEM((CHUNK,), jnp.float32) for _ in range(NBUF)]
          + [pltpu.VMEM((CHUNK, 32), jnp.float32) for _ in range(NBUF)]
          + [pltpu.VMEM_SHARED((N, 32), jnp.float32)]
          + [pltpu.SemaphoreType.DMA for _ in range(3 * NBUF + 1)]
      ),
  )
  def spmm(dst_hbm, src_hbm, eval_hbm, x_hbm, out_hbm, *scratch):
    src_v = scratch[0:NBUF]
    dst_v = scratch[NBUF:2 * NBUF]
    val_v = scratch[2 * NBUF:3 * NBUF]
    rows_v = scratch[3 * NBUF:4 * NBUF]
    acc_sh = scratch[4 * NBUF]
    semi = scratch[4 * NBUF + 1:4 * NBUF + 1 + NBUF]
    semg = scratch[4 * NBUF + 1 + NBUF:4 * NBUF + 1 + 2 * NBUF]
    sems = scratch[4 * NBUF + 1 + 2 * NBUF:4 * NBUF + 1 + 3 * NBUF]

    c = lax.axis_index("c")
    s = lax.axis_index("s")

    @plsc.parallel_loop(0, CHUNK)
    def _(i):
      rows_v[0][i, pl.ds(0, L)] = jnp.zeros((L,), jnp.float32)
      rows_v[0][i, pl.ds(L, L)] = jnp.zeros((L,), jnp.float32)

    zsem = scratch[4 * NBUF + 1 + 3 * NBUF]
    for z in range(ROWS_PER_TILE // CHUNK):
      pltpu.async_copy(rows_v[0],
                       acc_sh.at[pl.ds(s * ROWS_PER_TILE + z * CHUNK, CHUNK)],
                       zsem)
    zrem = ROWS_PER_TILE % CHUNK
    if zrem:
      pltpu.async_copy(
          rows_v[0].at[pl.ds(0, zrem)],
          acc_sh.at[pl.ds(s * ROWS_PER_TILE + (ROWS_PER_TILE // CHUNK) * CHUNK, zrem)],
          zsem)
    for z in range(ROWS_PER_TILE // CHUNK):
      pltpu.make_async_copy(
          rows_v[0],
          acc_sh.at[pl.ds(s * ROWS_PER_TILE + z * CHUNK, CHUNK)], zsem).wait()
    if zrem:
      pltpu.make_async_copy(
          rows_v[0].at[pl.ds(0, zrem)],
          acc_sh.at[pl.ds(s * ROWS_PER_TILE + (ROWS_PER_TILE // CHUNK) * CHUNK, zrem)],
          zsem).wait()
    plsc.subcore_barrier()

    if colsplit:
      tile_base = s * edges_per_tile
    else:
      tile_base = (c * NS + s) * edges_per_tile

    def fire_idx(g, b):
      e0 = tile_base + g * CHUNK
      pltpu.async_copy(src_hbm.at[pl.ds(e0, CHUNK)], src_v[b], semi[b])
      pltpu.async_copy(eval_hbm.at[pl.ds(e0, CHUNK)], val_v[b], semi[b])
      for j in range(NSUB):
        pltpu.async_copy(dst_hbm.at[pl.ds(e0 + j * SUB, SUB)], dst_v[b].at[j],
                         semi[b])

    def wait_idx(b):
      pltpu.make_async_copy(src_hbm.at[pl.ds(0, CHUNK)], src_v[b], semi[b]).wait()
      pltpu.make_async_copy(eval_hbm.at[pl.ds(0, CHUNK)], val_v[b], semi[b]).wait()
      for j in range(NSUB):
        pltpu.make_async_copy(dst_hbm.at[pl.ds(0, SUB)], dst_v[b].at[j],
                              semi[b]).wait()

    def transform(b):
      if colsplit:
        @plsc.parallel_loop(0, CHUNK, step=L)
        def _(k):
          src_v[b][pl.ds(k, L)] = src_v[b][pl.ds(k, L)] * 2 + c

    def fire_gather(b):
      for j in range(NSUB):
        pltpu.async_copy(x_hbm.at[src_v[b].at[pl.ds(j * SUB, SUB)]],
                         rows_v[b].at[pl.ds(j * SUB, SUB)], semg[b])

    def wait_gather(b):
      for j in range(NSUB):
        pltpu.make_async_copy(x_hbm.at[src_v[b].at[pl.ds(j * SUB, SUB)]],
                              rows_v[b].at[pl.ds(j * SUB, SUB)], semg[b]).wait()

    def scale(b):
      @plsc.parallel_loop(0, CHUNK, step=L)
      def _(g):
        vv = val_v[b][pl.ds(g, L)]
        for e0 in range(L):
          sp = jnp.broadcast_to(vv[e0], (L,))
          e = g + e0
          rows_v[b][e, pl.ds(0, L)] = rows_v[b][e, pl.ds(0, L)] * sp
          rows_v[b][e, pl.ds(L, L)] = rows_v[b][e, pl.ds(L, L)] * sp

    def fire_scatter(b):
      for j in range(NSUB):
        pltpu.async_copy(rows_v[b].at[pl.ds(j * SUB, SUB)],
                         acc_sh.at[dst_v[b].at[j]], sems[b], add=True)

    def wait_scatter(b):
      for j in range(NSUB):
        pltpu.make_async_copy(rows_v[b].at[pl.ds(j * SUB, SUB)],
                              acc_sh.at[dst_v[b].at[j]], sems[b]).wait()

    fire_idx(jnp.int32(0), 0)
    fire_idx(jnp.int32(1), 1)
    wait_idx(0)
    transform(0)
    fire_gather(0)
    wait_idx(1)
    transform(1)
    fire_gather(1)
    fire_idx(jnp.int32(2), 2)

    def group_body(g0, _):
      for b in range(NBUF):
        g = g0 * NBUF + b
        b2 = (b + 2) % NBUF
        b3 = (b + 3) % NBUF
        wait_gather(b)
        scale(b)
        fire_scatter(b)

        @pl.when(g + 2 < nchunk)
        def _():
          wait_idx(b2)
          transform(b2)

        @pl.when(g >= 1)
        def _():
          wait_scatter(b3)

        @pl.when(g + 3 < nchunk)
        def _():
          fire_idx(g + 3, b3)

        @pl.when(g + 2 < nchunk)
        def _():
          fire_gather(b2)
      return 0

    lax.fori_loop(0, nchunk // NBUF, group_body, 0)
    wait_scatter((nchunk - 1) % NBUF)

    plsc.subcore_barrier()
    @pl.when(s < NS - 1)
    def _():
      pltpu.sync_copy(acc_sh.at[pl.ds(s * OUT_RPT, OUT_RPT)],
                      out_hbm.at[pl.ds(c * N + s * OUT_RPT, OUT_RPT)])

    @pl.when(s == NS - 1)
    def _():
      pltpu.sync_copy(acc_sh.at[pl.ds((NS - 1) * OUT_RPT, OUT_LAST)],
                      out_hbm.at[pl.ds(c * N + (NS - 1) * OUT_RPT, OUT_LAST)])

  return spmm


_spmm_col = _spmm_sc(True)
_spmm_edge = _spmm_sc(False)


def _leaky(x):
  return jnp.where(x >= 0, x, 0.01 * x)


def _bi_combine(ego, side, W1, b1, W2, b2):
  s1 = _leaky(jnp.dot(ego + side, W1, preferred_element_type=jnp.float32) + b1)
  s2 = _leaky(jnp.dot(ego * side, W2, preferred_element_type=jnp.float32) + b2)
  new = s1 + s2
  nrm = jnp.sqrt(jnp.sum(new * new, axis=-1, keepdims=True))
  return new, new / jnp.maximum(nrm, 1e-12)


BN = 2000


def _combine_a_body(ego_ref, side_ref, w1_ref, b1_ref, w2_ref, b2_ref,
                    nxt_ref, nrm_ref):
  ego = ego_ref[...].reshape(BN, 64)
  side = jnp.concatenate([side_ref[0], side_ref[1]], axis=-1)
  new, nn = _bi_combine(ego, side, w1_ref[...], b1_ref[...], w2_ref[...], b2_ref[...])
  nxt_ref[...] = new.reshape(BN, 2, 32)
  nrm_ref[...] = nn


def _combine_b_body(ego_ref, side_ref, w1_ref, b1_ref, w2_ref, b2_ref,
                    nxt_ref):
  ego = ego_ref[...].reshape(BN, 64)
  side = jnp.concatenate([side_ref[0], side_ref[1]], axis=-1)
  new, _ = _bi_combine(ego, side, w1_ref[...], b1_ref[...], w2_ref[...], b2_ref[...])
  nxt_ref[...] = new


def _combine_c_body(ego0_ref, nrm1_ref, ego2_ref, side_ref,
                    w1_ref, b1_ref, w2_ref, b2_ref, out_ref):
  ego2 = ego2_ref[...]
  n2 = jnp.sqrt(jnp.sum(ego2 * ego2, axis=-1, keepdims=True))
  nrm2 = ego2 / jnp.maximum(n2, 1e-12)
  side = side_ref[0] + side_ref[1]
  _, nrm3 = _bi_combine(ego2, side, w1_ref[...], b1_ref[...], w2_ref[...], b2_ref[...])
  out_ref[...] = jnp.concatenate([ego0_ref[...], nrm1_ref[...], nrm2, nrm3],
                                 axis=-1)


def _il_spec():
  return pl.BlockSpec((BN, 2, 32), lambda i: (i, 0, 0))


def _split_spec():
  return pl.BlockSpec((2, BN, 32), lambda i: (0, i, 0))


def _full_spec(d):
  return pl.BlockSpec((BN, d), lambda i: (i, 0))


def _w_spec(din, dout):
  return pl.BlockSpec((din, dout), lambda i: (0, 0))


def _b_spec(dout):
  return pl.BlockSpec((1, dout), lambda i: (0, 0))


def _combine_a(ego_il, side_sp, W1, b1, W2, b2):
  return pl.pallas_call(
      _combine_a_body,
      grid=(N // BN,),
      in_specs=[_il_spec(), _split_spec(), _w_spec(64, 64), _b_spec(64),
                _w_spec(64, 64), _b_spec(64)],
      out_specs=[_il_spec(), _full_spec(64)],
      out_shape=[jax.ShapeDtypeStruct((N, 2, 32), jnp.float32),
                 jax.ShapeDtypeStruct((N, 64), jnp.float32)],
  )(ego_il, side_sp, W1, b1.reshape(1, -1), W2, b2.reshape(1, -1))


def _combine_b(ego_il, side_sp, W1, b1, W2, b2):
  return pl.pallas_call(
      _combine_b_body,
      grid=(N // BN,),
      in_specs=[_il_spec(), _split_spec(), _w_spec(64, 32), _b_spec(32),
                _w_spec(64, 32), _b_spec(32)],
      out_specs=_full_spec(32),
      out_shape=jax.ShapeDtypeStruct((N, 32), jnp.float32),
  )(ego_il, side_sp, W1, b1.reshape(1, -1), W2, b2.reshape(1, -1))


def _combine_c(ego0, nrm1, ego2, side_sp, W1, b1, W2, b2):
  return pl.pallas_call(
      _combine_c_body,
      grid=(N // BN,),
      in_specs=[_full_spec(64), _full_spec(64), _full_spec(32), _split_spec(),
                _w_spec(32, 16), _b_spec(16), _w_spec(32, 16), _b_spec(16)],
      out_specs=_full_spec(176),
      out_shape=jax.ShapeDtypeStruct((N, 176), jnp.float32),
  )(ego0, nrm1, ego2, side_sp, W1, b1.reshape(1, -1), W2, b2.reshape(1, -1))


def kernel(ego_embeddings, edge_index, edge_values,
           W1_0, b1_0, W2_0, b2_0,
           W1_1, b1_1, W2_1, b2_1,
           W1_2, b1_2, W2_2, b2_2):
  dst = jnp.pad(edge_index[0], (0, EP - E))
  src = jnp.pad(edge_index[1], (0, EP - E))
  ev = jnp.pad(edge_values, (0, EP - E))

  x0 = ego_embeddings.reshape(2 * N, 32)
  side0 = _spmm_col(dst, src, ev, x0)
  ego1_il, norm1 = _combine_a(ego_embeddings.reshape(N, 2, 32),
                              side0.reshape(2, N, 32), W1_0, b1_0, W2_0, b2_0)

  side1 = _spmm_col(dst, src, ev, ego1_il.reshape(2 * N, 32))
  ego2 = _combine_b(ego1_il, side1.reshape(2, N, 32), W1_1, b1_1, W2_1, b2_1)

  side2 = _spmm_edge(dst, src, ev, ego2)
  return _combine_c(ego_embeddings, norm1, ego2, side2.reshape(2, N, 32),
                    W1_2, b1_2, W2_2, b2_2)
